# Initial kernel scaffold; baseline (speedup 1.0000x reference)
import jax
import jax.numpy as jnp
from jax.experimental import pallas as pl


def kernel(x, edge_attr, gga, edge_index, params):
    raise NotImplementedError("write your pallas kernel here")


# speedup vs baseline: 7.6761x; 7.6761x over previous
import functools

import jax
import jax.numpy as jnp
from jax.experimental import pallas as pl

B, T, NN, NE, D = 16, 32, 16, 240, 64
G = B * T
GB = 8
NBLK = G // GB
EB = GB * NE
NB = GB * NN
F_IN = 90
E_IN = 4


def _mm_nt(a, b):
    return jax.lax.dot_general(a, b, (((1,), (1,)), ((), ())),
                               preferred_element_type=jnp.float32)


def _mm_tn(a, b):
    return jax.lax.dot_general(a, b, (((0,), (0,)), ((), ())),
                               preferred_element_type=jnp.float32)


def _mm_nn(a, b):
    return jax.lax.dot_general(a, b, (((1,), (0,)), ((), ())),
                               preferred_element_type=jnp.float32)


def _prep_body(gga_ref, w1_ref, b1_ref, w2_ref, b2_ref, pe1_ref, pn1_ref,
               gall_ref, ge1_ref, gn1_ref):
    h = _mm_nt(gga_ref[...], w1_ref[...]) + b1_ref[...]
    h = jnp.where(h >= 0, h, 0.01 * h)
    g_all = _mm_nt(h, w2_ref[...]) + b2_ref[...]
    gall_ref[...] = g_all
    ge1_ref[...] = _mm_nt(g_all, pe1_ref[...])
    gn1_ref[...] = _mm_nt(g_all, pn1_ref[...])


def _mp1_body(x_ref, ea_ref, sst_ref, sdt_ref, poole_ref, pooln_ref,
              gblk_ref, ge1_ref, gn1_ref,
              wn_ref, bn_ref, we_ref, be_ref,
              wpi_ref, wpj_ref, wpe_ref, pb1_ref, pw2_ref, pb2_ref,
              wgx_ref, wga_ref, gb1_ref, gw2_ref, gb2_ref,
              wno_ref, weo_ref, wgo_ref, ob1_ref, ow2_ref, ob2_ref,
              pe2_ref, pn2_ref,
              phi_out, gam_out, gga_out, ge2_out, gn2_out):
    i = pl.program_id(0)
    xe = _mm_nt(x_ref[...], wn_ref[...]) + bn_ref[...]
    ef = _mm_nt(ea_ref[...], we_ref[...]) + be_ref[...]
    p_i = _mm_nt(xe, wpi_ref[...])
    p_j = _mm_nt(xe, wpj_ref[...])
    e_t = _mm_nt(ef, wpe_ref[...])
    st_e = pl.multiple_of(jax.lax.rem(i * EB, G), NB)
    st_n = pl.multiple_of(jax.lax.rem(i * NB, G), NB)
    ge_win = ge1_ref[pl.ds(st_e, EB), :]
    gn_win = gn1_ref[pl.ds(st_n, NB), :]
    pre = (_mm_tn(sdt_ref[...], p_i) + _mm_tn(sst_ref[...], p_j)
           + e_t + ge_win + pb1_ref[...])
    h = jnp.maximum(pre, 0.0)
    phi = _mm_nt(h, pw2_ref[...]) + pb2_ref[...]
    phi_out[...] = phi
    agg = _mm_nn(sdt_ref[...], phi)
    gpre = (_mm_nt(xe, wgx_ref[...]) + _mm_nt(agg, wga_ref[...])
            + gn_win + gb1_ref[...])
    gam = _mm_nt(jnp.maximum(gpre, 0.0), gw2_ref[...]) + gb2_ref[...]
    gam_out[...] = gam
    npool = _mm_nn(pooln_ref[...], gam)
    epool = _mm_nn(poole_ref[...], phi)
    opre = (_mm_nt(npool, wno_ref[...]) + _mm_nt(epool, weo_ref[...])
            + _mm_nt(gblk_ref[...], wgo_ref[...]) + ob1_ref[...])
    gga1 = _mm_nt(jnp.maximum(opre, 0.0), ow2_ref[...]) + ob2_ref[...]
    gga_out[...] = gga1
    ge2_out[...] = _mm_nt(gga1, pe2_ref[...])
    gn2_out[...] = _mm_nt(gga1, pn2_ref[...])


def _mp2_body(xn_ref, xe_ref, sst_ref, sdt_ref,
              ge2_ref, gn2_ref,
              wpi_ref, wpj_ref, wpe_ref, pb1_ref, pw2_ref, pb2_ref,
              wgx_ref, wga_ref, gb1_ref, gw2_ref, gb2_ref,
              gam_out):
    i = pl.program_id(0)
    xn = xn_ref[...]
    p_i = _mm_nt(xn, wpi_ref[...])
    p_j = _mm_nt(xn, wpj_ref[...])
    e_t = _mm_nt(xe_ref[...], wpe_ref[...])
    st_e = pl.multiple_of(jax.lax.rem(i * EB, G), NB)
    st_n = pl.multiple_of(jax.lax.rem(i * NB, G), NB)
    ge_win = ge2_ref[pl.ds(st_e, EB), :]
    gn_win = gn2_ref[pl.ds(st_n, NB), :]
    pre = (_mm_tn(sdt_ref[...], p_i) + _mm_tn(sst_ref[...], p_j)
           + e_t + ge_win + pb1_ref[...])
    h = jnp.maximum(pre, 0.0)
    phi = _mm_nt(h, pw2_ref[...]) + pb2_ref[...]
    agg = _mm_nn(sdt_ref[...], phi)
    gpre = (_mm_nt(xn, wgx_ref[...]) + _mm_nt(agg, wga_ref[...])
            + gn_win + gb1_ref[...])
    gam_out[...] = _mm_nt(jnp.maximum(gpre, 0.0), gw2_ref[...]) + gb2_ref[...]


def _lstm_body(seq_ref, wih_ref, whh_ref, bias_ref, h_out):
    bn = NN * B

    def step(t, carry):
        h, c = carry
        xt = seq_ref[t]
        gates = _mm_nt(xt, wih_ref[...]) + _mm_nt(h, whh_ref[...]) + bias_ref[...]
        ig = jax.nn.sigmoid(gates[:, 0:D])
        fg = jax.nn.sigmoid(gates[:, D:2 * D])
        gg = jnp.tanh(gates[:, 2 * D:3 * D])
        og = jax.nn.sigmoid(gates[:, 3 * D:4 * D])
        c = fg * c + ig * gg
        h = og * jnp.tanh(c)
        return (h, c)

    init = (jnp.zeros((bn, D), jnp.float32), jnp.zeros((bn, D), jnp.float32))
    h, _ = jax.lax.fori_loop(0, T, step, init)
    h_out[...] = h


def _full(shape):
    nd = len(shape)
    return pl.BlockSpec(shape, lambda i: (0,) * nd)


def kernel(x, edge_attr, gga, edge_index, params):
    f32 = jnp.float32
    xflat = x.reshape(G * NN, F_IN).astype(f32)
    eaflat = edge_attr.reshape(G * NE, E_IN).astype(f32)

    src = edge_index[0].astype(jnp.int32)
    dst = edge_index[1].astype(jnp.int32)
    off = (jnp.arange(GB, dtype=jnp.int32) * NN)[:, None]
    srcb = (src[None, :] + off).reshape(EB)
    dstb = (dst[None, :] + off).reshape(EB)
    lanes = jnp.arange(NB, dtype=jnp.int32)[:, None]
    sst = (srcb[None, :] == lanes).astype(f32)
    sdt = (dstb[None, :] == lanes).astype(f32)
    poole = ((jnp.arange(EB) // NE)[None, :]
             == jnp.arange(GB)[:, None]).astype(f32) / NE
    pooln = ((jnp.arange(NB) // NN)[None, :]
             == jnp.arange(GB)[:, None]).astype(f32) / NN

    wn, bn_b = params['node_emb']
    we, be_b = params['edge_emb']
    w1g, b1g = params['gga1']
    w2g, b2g = params['gga2']

    def mp_parts(p):
        (pw1, pb1), (pw2, pb2) = p['phi']
        (gw1, gb1), (gw2, gb2) = p['gamma']
        (ow1, ob1), (ow2, ob2) = p['phi_global']
        return dict(
            wpi=pw1[:, 0:D], wpj=pw1[:, D:2 * D], wpe=pw1[:, 2 * D:3 * D],
            wpg=pw1[:, 3 * D:4 * D], pb1=pb1[None], pw2=pw2, pb2=pb2[None],
            wgx=gw1[:, 0:D], wga=gw1[:, D:2 * D], wgg=gw1[:, 2 * D:3 * D],
            gb1=gb1[None], gw2=gw2, gb2=gb2[None],
            wno=ow1[:, 0:D], weo=ow1[:, D:2 * D], wgo=ow1[:, 2 * D:3 * D],
            ob1=ob1[None], ow2=ow2, ob2=ob2[None])

    m1 = mp_parts(params['mp1'])
    m2 = mp_parts(params['mp2'])

    g_all, ge1, gn1 = pl.pallas_call(
        _prep_body,
        grid=(1,),
        in_specs=[_full((G, 32)), _full((256, 32)), _full((1, 256)),
                  _full((D, 256)), _full((1, D)), _full((D, D)), _full((D, D))],
        out_specs=[_full((G, D)), _full((G, D)), _full((G, D))],
        out_shape=[jax.ShapeDtypeStruct((G, D), f32)] * 3,
    )(gga.astype(f32), w1g, b1g[None], w2g, b2g[None], m1['wpg'], m1['wgg'])

    ge1t = jnp.concatenate([ge1] * 5, axis=0)

    row2 = lambda i: (i, 0)
    phi1, gam1, gga1, ge2, gn2 = pl.pallas_call(
        _mp1_body,
        grid=(NBLK,),
        in_specs=[
            pl.BlockSpec((NB, F_IN), row2),
            pl.BlockSpec((EB, E_IN), row2),
            _full((NB, EB)), _full((NB, EB)),
            _full((GB, EB)), _full((GB, NB)),
            pl.BlockSpec((GB, D), row2),
            _full((5 * G, D)), _full((G, D)),
            _full((D, F_IN)), _full((1, D)), _full((D, E_IN)), _full((1, D)),
            _full((D, D)), _full((D, D)), _full((D, D)), _full((1, D)),
            _full((D, D)), _full((1, D)),
            _full((D, D)), _full((D, D)), _full((1, D)), _full((D, D)),
            _full((1, D)),
            _full((D, D)), _full((D, D)), _full((D, D)), _full((1, D)),
            _full((D, D)), _full((1, D)),
            _full((D, D)), _full((D, D)),
        ],
        out_specs=[
            pl.BlockSpec((EB, D), row2),
            pl.BlockSpec((NB, D), row2),
            pl.BlockSpec((GB, D), row2),
            pl.BlockSpec((GB, D), row2),
            pl.BlockSpec((GB, D), row2),
        ],
        out_shape=[
            jax.ShapeDtypeStruct((G * NE, D), f32),
            jax.ShapeDtypeStruct((G * NN, D), f32),
            jax.ShapeDtypeStruct((G, D), f32),
            jax.ShapeDtypeStruct((G, D), f32),
            jax.ShapeDtypeStruct((G, D), f32),
        ],
    )(xflat, eaflat, sst, sdt, poole, pooln, g_all, ge1t, gn1,
      wn, bn_b[None], we, be_b[None],
      m1['wpi'], m1['wpj'], m1['wpe'], m1['pb1'], m1['pw2'], m1['pb2'],
      m1['wgx'], m1['wga'], m1['gb1'], m1['gw2'], m1['gb2'],
      m1['wno'], m1['weo'], m1['wgo'], m1['ob1'], m1['ow2'], m1['ob2'],
      m2['wpg'], m2['wgg'])

    ge2t = jnp.concatenate([ge2] * 5, axis=0)

    gam2 = pl.pallas_call(
        _mp2_body,
        grid=(NBLK,),
        in_specs=[
            pl.BlockSpec((NB, D), row2),
            pl.BlockSpec((EB, D), row2),
            _full((NB, EB)), _full((NB, EB)),
            _full((5 * G, D)), _full((G, D)),
            _full((D, D)), _full((D, D)), _full((D, D)), _full((1, D)),
            _full((D, D)), _full((1, D)),
            _full((D, D)), _full((D, D)), _full((1, D)), _full((D, D)),
            _full((1, D)),
        ],
        out_specs=[pl.BlockSpec((NB, D), row2)],
        out_shape=[jax.ShapeDtypeStruct((G * NN, D), f32)],
    )(gam1, phi1, sst, sdt, ge2t, gn2,
      m2['wpi'], m2['wpj'], m2['wpe'], m2['pb1'], m2['pw2'], m2['pb2'],
      m2['wgx'], m2['wga'], m2['gb1'], m2['gw2'], m2['gb2'])[0]

    lp = params['lstm']
    seq = gam2.reshape(T, NN * B, D)
    bias = (lp['bih'] + lp['bhh'])[None]
    h = pl.pallas_call(
        _lstm_body,
        grid=(1,),
        in_specs=[_full((T, NN * B, D)), _full((4 * D, D)), _full((4 * D, D)),
                  _full((1, 4 * D))],
        out_specs=[_full((NN * B, D))],
        out_shape=[jax.ShapeDtypeStruct((NN * B, D), f32)],
    )(seq, lp['Wih'], lp['Whh'], bias)[0]

    return h.reshape(B, NN, D)


# speedup vs baseline: 7.6911x; 1.0020x over previous
import functools

import jax
import jax.numpy as jnp
from jax.experimental import pallas as pl

B, T, NN, NE, D = 16, 32, 16, 240, 64
G = B * T
GB = 8
NBLK = G // GB
EB = GB * NE
NB = GB * NN
F_IN = 90
E_IN = 4


def _mm_nt(a, b):
    return jax.lax.dot_general(a, b, (((1,), (1,)), ((), ())),
                               preferred_element_type=jnp.float32)


def _mm_tn(a, b):
    return jax.lax.dot_general(a, b, (((0,), (0,)), ((), ())),
                               preferred_element_type=jnp.float32)


def _mm_nn(a, b):
    return jax.lax.dot_general(a, b, (((1,), (0,)), ((), ())),
                               preferred_element_type=jnp.float32)


def _prep_body(gga_ref, w1_ref, b1_ref, w2_ref, b2_ref, pe1_ref, pn1_ref,
               gall_ref, ge1_ref, gn1_ref):
    h = _mm_nt(gga_ref[...], w1_ref[...]) + b1_ref[...]
    h = jnp.where(h >= 0, h, 0.01 * h)
    g_all = _mm_nt(h, w2_ref[...]) + b2_ref[...]
    gall_ref[...] = g_all
    ge1_ref[...] = _mm_nt(g_all, pe1_ref[...])
    gn1_ref[...] = _mm_nt(g_all, pn1_ref[...])


def _mp1_body(x_ref, ea_ref, ss_ref, sd_ref, sdt_ref, poole_ref, pooln_ref,
              gblk_ref, ge1_ref, gn1_ref,
              wn_ref, bn_ref, we_ref, be_ref,
              wpi_ref, wpj_ref, wpe_ref, pb1_ref, pw2_ref, pb2_ref,
              wgx_ref, wga_ref, gb1_ref, gw2_ref, gb2_ref,
              wno_ref, weo_ref, wgo_ref, ob1_ref, ow2_ref, ob2_ref,
              pe2_ref, pn2_ref,
              phi_out, gam_out, gga_out, ge2_out, gn2_out):
    i = pl.program_id(0)
    xe = _mm_nt(x_ref[...], wn_ref[...]) + bn_ref[...]
    ef = _mm_nt(ea_ref[...], we_ref[...]) + be_ref[...]
    p_i = _mm_nt(xe, wpi_ref[...])
    p_j = _mm_nt(xe, wpj_ref[...])
    e_t = _mm_nt(ef, wpe_ref[...])
    st_e = pl.multiple_of(jax.lax.rem(i * EB, G), NB)
    st_n = pl.multiple_of(jax.lax.rem(i * NB, G), NB)
    ge_win = ge1_ref[pl.ds(st_e, EB), :]
    gn_win = gn1_ref[pl.ds(st_n, NB), :]
    pre = (_mm_nn(sd_ref[...], p_i) + _mm_nn(ss_ref[...], p_j)
           + e_t + ge_win + pb1_ref[...])
    h = jnp.maximum(pre, 0.0)
    phi = _mm_nt(h, pw2_ref[...]) + pb2_ref[...]
    phi_out[...] = phi
    agg = _mm_nn(sdt_ref[...], phi)
    gpre = (_mm_nt(xe, wgx_ref[...]) + _mm_nt(agg, wga_ref[...])
            + gn_win + gb1_ref[...])
    gam = _mm_nt(jnp.maximum(gpre, 0.0), gw2_ref[...]) + gb2_ref[...]
    gam_out[...] = gam
    npool = _mm_nn(pooln_ref[...], gam)
    epool = _mm_nn(poole_ref[...], phi)
    opre = (_mm_nt(npool, wno_ref[...]) + _mm_nt(epool, weo_ref[...])
            + _mm_nt(gblk_ref[...], wgo_ref[...]) + ob1_ref[...])
    gga1 = _mm_nt(jnp.maximum(opre, 0.0), ow2_ref[...]) + ob2_ref[...]
    gga_out[...] = gga1
    ge2_out[...] = _mm_nt(gga1, pe2_ref[...])
    gn2_out[...] = _mm_nt(gga1, pn2_ref[...])


def _mp2_body(xn_ref, xe_ref, ss_ref, sd_ref, sdt_ref,
              ge2_ref, gn2_ref,
              wpi_ref, wpj_ref, wpe_ref, pb1_ref, pw2_ref, pb2_ref,
              wgx_ref, wga_ref, gb1_ref, gw2_ref, gb2_ref,
              gam_out):
    i = pl.program_id(0)
    xn = xn_ref[...]
    p_i = _mm_nt(xn, wpi_ref[...])
    p_j = _mm_nt(xn, wpj_ref[...])
    e_t = _mm_nt(xe_ref[...], wpe_ref[...])
    st_e = pl.multiple_of(jax.lax.rem(i * EB, G), NB)
    st_n = pl.multiple_of(jax.lax.rem(i * NB, G), NB)
    ge_win = ge2_ref[pl.ds(st_e, EB), :]
    gn_win = gn2_ref[pl.ds(st_n, NB), :]
    pre = (_mm_nn(sd_ref[...], p_i) + _mm_nn(ss_ref[...], p_j)
           + e_t + ge_win + pb1_ref[...])
    h = jnp.maximum(pre, 0.0)
    phi = _mm_nt(h, pw2_ref[...]) + pb2_ref[...]
    agg = _mm_nn(sdt_ref[...], phi)
    gpre = (_mm_nt(xn, wgx_ref[...]) + _mm_nt(agg, wga_ref[...])
            + gn_win + gb1_ref[...])
    gam_out[...] = _mm_nt(jnp.maximum(gpre, 0.0), gw2_ref[...]) + gb2_ref[...]


def _lstm_body(seq_ref, wih_ref, whh_ref, bias_ref, h_out):
    bn = NN * B

    def step(t, carry):
        h, c = carry
        xt = seq_ref[t]
        gates = _mm_nt(xt, wih_ref[...]) + _mm_nt(h, whh_ref[...]) + bias_ref[...]
        ig = jax.nn.sigmoid(gates[:, 0:D])
        fg = jax.nn.sigmoid(gates[:, D:2 * D])
        gg = jnp.tanh(gates[:, 2 * D:3 * D])
        og = jax.nn.sigmoid(gates[:, 3 * D:4 * D])
        c = fg * c + ig * gg
        h = og * jnp.tanh(c)
        return (h, c)

    init = (jnp.zeros((bn, D), jnp.float32), jnp.zeros((bn, D), jnp.float32))
    h, _ = jax.lax.fori_loop(0, T, step, init)
    h_out[...] = h


def _full(shape):
    nd = len(shape)
    return pl.BlockSpec(shape, lambda i: (0,) * nd)


def kernel(x, edge_attr, gga, edge_index, params):
    f32 = jnp.float32
    xflat = x.reshape(G * NN, F_IN).astype(f32)
    eaflat = edge_attr.reshape(G * NE, E_IN).astype(f32)

    src = edge_index[0].astype(jnp.int32)
    dst = edge_index[1].astype(jnp.int32)
    off = (jnp.arange(GB, dtype=jnp.int32) * NN)[:, None]
    srcb = (src[None, :] + off).reshape(EB)
    dstb = (dst[None, :] + off).reshape(EB)
    lanes = jnp.arange(NB, dtype=jnp.int32)[:, None]
    sdt = (dstb[None, :] == lanes).astype(f32)
    ss = (srcb[:, None] == lanes.T).astype(f32)
    sd = (dstb[:, None] == lanes.T).astype(f32)
    poole = ((jnp.arange(EB) // NE)[None, :]
             == jnp.arange(GB)[:, None]).astype(f32) / NE
    pooln = ((jnp.arange(NB) // NN)[None, :]
             == jnp.arange(GB)[:, None]).astype(f32) / NN

    wn, bn_b = params['node_emb']
    we, be_b = params['edge_emb']
    w1g, b1g = params['gga1']
    w2g, b2g = params['gga2']

    def mp_parts(p):
        (pw1, pb1), (pw2, pb2) = p['phi']
        (gw1, gb1), (gw2, gb2) = p['gamma']
        (ow1, ob1), (ow2, ob2) = p['phi_global']
        return dict(
            wpi=pw1[:, 0:D], wpj=pw1[:, D:2 * D], wpe=pw1[:, 2 * D:3 * D],
            wpg=pw1[:, 3 * D:4 * D], pb1=pb1[None], pw2=pw2, pb2=pb2[None],
            wgx=gw1[:, 0:D], wga=gw1[:, D:2 * D], wgg=gw1[:, 2 * D:3 * D],
            gb1=gb1[None], gw2=gw2, gb2=gb2[None],
            wno=ow1[:, 0:D], weo=ow1[:, D:2 * D], wgo=ow1[:, 2 * D:3 * D],
            ob1=ob1[None], ow2=ow2, ob2=ob2[None])

    m1 = mp_parts(params['mp1'])
    m2 = mp_parts(params['mp2'])

    g_all, ge1, gn1 = pl.pallas_call(
        _prep_body,
        grid=(1,),
        in_specs=[_full((G, 32)), _full((256, 32)), _full((1, 256)),
                  _full((D, 256)), _full((1, D)), _full((D, D)), _full((D, D))],
        out_specs=[_full((G, D)), _full((G, D)), _full((G, D))],
        out_shape=[jax.ShapeDtypeStruct((G, D), f32)] * 3,
    )(gga.astype(f32), w1g, b1g[None], w2g, b2g[None], m1['wpg'], m1['wgg'])

    ge1t = jnp.concatenate([ge1] * 5, axis=0)

    row2 = lambda i: (i, 0)
    phi1, gam1, gga1, ge2, gn2 = pl.pallas_call(
        _mp1_body,
        grid=(NBLK,),
        in_specs=[
            pl.BlockSpec((NB, F_IN), row2),
            pl.BlockSpec((EB, E_IN), row2),
            _full((EB, NB)), _full((EB, NB)), _full((NB, EB)),
            _full((GB, EB)), _full((GB, NB)),
            pl.BlockSpec((GB, D), row2),
            _full((5 * G, D)), _full((G, D)),
            _full((D, F_IN)), _full((1, D)), _full((D, E_IN)), _full((1, D)),
            _full((D, D)), _full((D, D)), _full((D, D)), _full((1, D)),
            _full((D, D)), _full((1, D)),
            _full((D, D)), _full((D, D)), _full((1, D)), _full((D, D)),
            _full((1, D)),
            _full((D, D)), _full((D, D)), _full((D, D)), _full((1, D)),
            _full((D, D)), _full((1, D)),
            _full((D, D)), _full((D, D)),
        ],
        out_specs=[
            pl.BlockSpec((EB, D), row2),
            pl.BlockSpec((NB, D), row2),
            pl.BlockSpec((GB, D), row2),
            pl.BlockSpec((GB, D), row2),
            pl.BlockSpec((GB, D), row2),
        ],
        out_shape=[
            jax.ShapeDtypeStruct((G * NE, D), f32),
            jax.ShapeDtypeStruct((G * NN, D), f32),
            jax.ShapeDtypeStruct((G, D), f32),
            jax.ShapeDtypeStruct((G, D), f32),
            jax.ShapeDtypeStruct((G, D), f32),
        ],
    )(xflat, eaflat, ss, sd, sdt, poole, pooln, g_all, ge1t, gn1,
      wn, bn_b[None], we, be_b[None],
      m1['wpi'], m1['wpj'], m1['wpe'], m1['pb1'], m1['pw2'], m1['pb2'],
      m1['wgx'], m1['wga'], m1['gb1'], m1['gw2'], m1['gb2'],
      m1['wno'], m1['weo'], m1['wgo'], m1['ob1'], m1['ow2'], m1['ob2'],
      m2['wpg'], m2['wgg'])

    ge2t = jnp.concatenate([ge2] * 5, axis=0)

    gam2 = pl.pallas_call(
        _mp2_body,
        grid=(NBLK,),
        in_specs=[
            pl.BlockSpec((NB, D), row2),
            pl.BlockSpec((EB, D), row2),
            _full((EB, NB)), _full((EB, NB)), _full((NB, EB)),
            _full((5 * G, D)), _full((G, D)),
            _full((D, D)), _full((D, D)), _full((D, D)), _full((1, D)),
            _full((D, D)), _full((1, D)),
            _full((D, D)), _full((D, D)), _full((1, D)), _full((D, D)),
            _full((1, D)),
        ],
        out_specs=[pl.BlockSpec((NB, D), row2)],
        out_shape=[jax.ShapeDtypeStruct((G * NN, D), f32)],
    )(gam1, phi1, ss, sd, sdt, ge2t, gn2,
      m2['wpi'], m2['wpj'], m2['wpe'], m2['pb1'], m2['pw2'], m2['pb2'],
      m2['wgx'], m2['wga'], m2['gb1'], m2['gw2'], m2['gb2'])[0]

    lp = params['lstm']
    seq = gam2.reshape(T, NN * B, D)
    bias = (lp['bih'] + lp['bhh'])[None]
    h = pl.pallas_call(
        _lstm_body,
        grid=(1,),
        in_specs=[_full((T, NN * B, D)), _full((4 * D, D)), _full((4 * D, D)),
                  _full((1, 4 * D))],
        out_specs=[_full((NN * B, D))],
        out_shape=[jax.ShapeDtypeStruct((NN * B, D), f32)],
    )(seq, lp['Wih'], lp['Whh'], bias)[0]

    return h.reshape(B, NN, D)


# speedup vs baseline: 7.7948x; 1.0135x over previous
import functools

import jax
import jax.numpy as jnp
from jax.experimental import pallas as pl

B, T, NN, NE, D = 16, 32, 16, 240, 64
G = B * T
GB = 8
NBLK = G // GB
EB = GB * NE
NB = GB * NN
F_IN = 90
E_IN = 4


def _mm_nt(a, b):
    return jax.lax.dot_general(a, b, (((1,), (1,)), ((), ())),
                               preferred_element_type=jnp.float32)


def _mm_tn(a, b):
    return jax.lax.dot_general(a, b, (((0,), (0,)), ((), ())),
                               preferred_element_type=jnp.float32)


def _mm_nn(a, b):
    return jax.lax.dot_general(a, b, (((1,), (0,)), ((), ())),
                               preferred_element_type=jnp.float32)


def _prep_body(gga_ref, w1_ref, b1_ref, w2_ref, b2_ref, pe1_ref, pn1_ref,
               gall_ref, ge1_ref, gn1_ref):
    h = _mm_nt(gga_ref[...], w1_ref[...]) + b1_ref[...]
    h = jnp.where(h >= 0, h, 0.01 * h)
    g_all = _mm_nt(h, w2_ref[...]) + b2_ref[...]
    gall_ref[...] = g_all
    ge1_ref[...] = _mm_nt(g_all, pe1_ref[...])
    gn1_ref[...] = _mm_nt(g_all, pn1_ref[...])


def _onehots(eib_ref):
    srow = eib_ref[0:1, :]
    drow = eib_ref[1:2, :]
    li = jax.lax.broadcasted_iota(jnp.int32, (NB, EB), 0)
    sst = (srow == li).astype(jnp.float32)
    sdt = (drow == li).astype(jnp.float32)
    return sst, sdt


def _mp1_body(x_ref, ea_ref, eib_ref, poole_ref, pooln_ref,
              gblk_ref, ge1_ref, gn1_ref,
              wn_ref, bn_ref, we_ref, be_ref,
              wpi_ref, wpj_ref, wpe_ref, pb1_ref, pw2_ref, pb2_ref,
              wgx_ref, wga_ref, gb1_ref, gw2_ref, gb2_ref,
              wno_ref, weo_ref, wgo_ref, ob1_ref, ow2_ref, ob2_ref,
              pe2_ref, pn2_ref,
              phi_out, gam_out, gga_out, ge2_out, gn2_out):
    i = pl.program_id(0)
    xe = _mm_nt(x_ref[...], wn_ref[...]) + bn_ref[...]
    ef = _mm_nt(ea_ref[...], we_ref[...]) + be_ref[...]
    p_i = _mm_nt(xe, wpi_ref[...])
    p_j = _mm_nt(xe, wpj_ref[...])
    e_t = _mm_nt(ef, wpe_ref[...])
    st_e = pl.multiple_of(jax.lax.rem(i * EB, G), NB)
    st_n = pl.multiple_of(jax.lax.rem(i * NB, G), NB)
    ge_win = ge1_ref[pl.ds(st_e, EB), :]
    gn_win = gn1_ref[pl.ds(st_n, NB), :]
    sst, sdt = _onehots(eib_ref)
    pre = (_mm_tn(sdt, p_i) + _mm_tn(sst, p_j)
           + e_t + ge_win + pb1_ref[...])
    h = jnp.maximum(pre, 0.0)
    phi = _mm_nt(h, pw2_ref[...]) + pb2_ref[...]
    phi_out[...] = phi
    agg = _mm_nn(sdt, phi)
    gpre = (_mm_nt(xe, wgx_ref[...]) + _mm_nt(agg, wga_ref[...])
            + gn_win + gb1_ref[...])
    gam = _mm_nt(jnp.maximum(gpre, 0.0), gw2_ref[...]) + gb2_ref[...]
    gam_out[...] = gam
    npool = _mm_nn(pooln_ref[...], gam)
    epool = _mm_nn(poole_ref[...], phi)
    opre = (_mm_nt(npool, wno_ref[...]) + _mm_nt(epool, weo_ref[...])
            + _mm_nt(gblk_ref[...], wgo_ref[...]) + ob1_ref[...])
    gga1 = _mm_nt(jnp.maximum(opre, 0.0), ow2_ref[...]) + ob2_ref[...]
    gga_out[...] = gga1
    ge2_out[...] = _mm_nt(gga1, pe2_ref[...])
    gn2_out[...] = _mm_nt(gga1, pn2_ref[...])


def _mp2_body(xn_ref, xe_ref, eib_ref,
              ge2_ref, gn2_ref,
              wpi_ref, wpj_ref, wpe_ref, pb1_ref, pw2_ref, pb2_ref,
              wgx_ref, wga_ref, gb1_ref, gw2_ref, gb2_ref,
              gam_out):
    i = pl.program_id(0)
    xn = xn_ref[...]
    p_i = _mm_nt(xn, wpi_ref[...])
    p_j = _mm_nt(xn, wpj_ref[...])
    e_t = _mm_nt(xe_ref[...], wpe_ref[...])
    st_e = pl.multiple_of(jax.lax.rem(i * EB, G), NB)
    st_n = pl.multiple_of(jax.lax.rem(i * NB, G), NB)
    ge_win = ge2_ref[pl.ds(st_e, EB), :]
    gn_win = gn2_ref[pl.ds(st_n, NB), :]
    sst, sdt = _onehots(eib_ref)
    pre = (_mm_tn(sdt, p_i) + _mm_tn(sst, p_j)
           + e_t + ge_win + pb1_ref[...])
    h = jnp.maximum(pre, 0.0)
    phi = _mm_nt(h, pw2_ref[...]) + pb2_ref[...]
    agg = _mm_nn(sdt, phi)
    gpre = (_mm_nt(xn, wgx_ref[...]) + _mm_nt(agg, wga_ref[...])
            + gn_win + gb1_ref[...])
    gam_out[...] = _mm_nt(jnp.maximum(gpre, 0.0), gw2_ref[...]) + gb2_ref[...]


def _lstm_body(seq_ref, wih_ref, whh_ref, bias_ref, h_out):
    bn = NN * B

    def step(t, carry):
        h, c = carry
        xt = seq_ref[t]
        gates = _mm_nt(xt, wih_ref[...]) + _mm_nt(h, whh_ref[...]) + bias_ref[...]
        ig = jax.nn.sigmoid(gates[:, 0:D])
        fg = jax.nn.sigmoid(gates[:, D:2 * D])
        gg = jnp.tanh(gates[:, 2 * D:3 * D])
        og = jax.nn.sigmoid(gates[:, 3 * D:4 * D])
        c = fg * c + ig * gg
        h = og * jnp.tanh(c)
        return (h, c)

    init = (jnp.zeros((bn, D), jnp.float32), jnp.zeros((bn, D), jnp.float32))
    h, _ = jax.lax.fori_loop(0, T, step, init)
    h_out[...] = h


def _full(shape):
    nd = len(shape)
    return pl.BlockSpec(shape, lambda i: (0,) * nd)


def kernel(x, edge_attr, gga, edge_index, params):
    f32 = jnp.float32
    xflat = x.reshape(G * NN, F_IN).astype(f32)
    eaflat = edge_attr.reshape(G * NE, E_IN).astype(f32)

    src = edge_index[0].astype(jnp.int32)
    dst = edge_index[1].astype(jnp.int32)
    off = (jnp.arange(GB, dtype=jnp.int32) * NN)[:, None]
    srcb = (src[None, :] + off).reshape(EB)
    dstb = (dst[None, :] + off).reshape(EB)
    eib = jnp.zeros((8, EB), jnp.int32).at[0].set(srcb).at[1].set(dstb)
    poole = ((jnp.arange(EB) // NE)[None, :]
             == jnp.arange(GB)[:, None]).astype(f32) / NE
    pooln = ((jnp.arange(NB) // NN)[None, :]
             == jnp.arange(GB)[:, None]).astype(f32) / NN

    wn, bn_b = params['node_emb']
    we, be_b = params['edge_emb']
    w1g, b1g = params['gga1']
    w2g, b2g = params['gga2']

    def mp_parts(p):
        (pw1, pb1), (pw2, pb2) = p['phi']
        (gw1, gb1), (gw2, gb2) = p['gamma']
        (ow1, ob1), (ow2, ob2) = p['phi_global']
        return dict(
            wpi=pw1[:, 0:D], wpj=pw1[:, D:2 * D], wpe=pw1[:, 2 * D:3 * D],
            wpg=pw1[:, 3 * D:4 * D], pb1=pb1[None], pw2=pw2, pb2=pb2[None],
            wgx=gw1[:, 0:D], wga=gw1[:, D:2 * D], wgg=gw1[:, 2 * D:3 * D],
            gb1=gb1[None], gw2=gw2, gb2=gb2[None],
            wno=ow1[:, 0:D], weo=ow1[:, D:2 * D], wgo=ow1[:, 2 * D:3 * D],
            ob1=ob1[None], ow2=ow2, ob2=ob2[None])

    m1 = mp_parts(params['mp1'])
    m2 = mp_parts(params['mp2'])

    g_all, ge1, gn1 = pl.pallas_call(
        _prep_body,
        grid=(1,),
        in_specs=[_full((G, 32)), _full((256, 32)), _full((1, 256)),
                  _full((D, 256)), _full((1, D)), _full((D, D)), _full((D, D))],
        out_specs=[_full((G, D)), _full((G, D)), _full((G, D))],
        out_shape=[jax.ShapeDtypeStruct((G, D), f32)] * 3,
    )(gga.astype(f32), w1g, b1g[None], w2g, b2g[None], m1['wpg'], m1['wgg'])

    ge1t = jnp.concatenate([ge1] * 5, axis=0)

    row2 = lambda i: (i, 0)
    phi1, gam1, gga1, ge2, gn2 = pl.pallas_call(
        _mp1_body,
        grid=(NBLK,),
        in_specs=[
            pl.BlockSpec((NB, F_IN), row2),
            pl.BlockSpec((EB, E_IN), row2),
            _full((8, EB)),
            _full((GB, EB)), _full((GB, NB)),
            pl.BlockSpec((GB, D), row2),
            _full((5 * G, D)), _full((G, D)),
            _full((D, F_IN)), _full((1, D)), _full((D, E_IN)), _full((1, D)),
            _full((D, D)), _full((D, D)), _full((D, D)), _full((1, D)),
            _full((D, D)), _full((1, D)),
            _full((D, D)), _full((D, D)), _full((1, D)), _full((D, D)),
            _full((1, D)),
            _full((D, D)), _full((D, D)), _full((D, D)), _full((1, D)),
            _full((D, D)), _full((1, D)),
            _full((D, D)), _full((D, D)),
        ],
        out_specs=[
            pl.BlockSpec((EB, D), row2),
            pl.BlockSpec((NB, D), row2),
            pl.BlockSpec((GB, D), row2),
            pl.BlockSpec((GB, D), row2),
            pl.BlockSpec((GB, D), row2),
        ],
        out_shape=[
            jax.ShapeDtypeStruct((G * NE, D), f32),
            jax.ShapeDtypeStruct((G * NN, D), f32),
            jax.ShapeDtypeStruct((G, D), f32),
            jax.ShapeDtypeStruct((G, D), f32),
            jax.ShapeDtypeStruct((G, D), f32),
        ],
    )(xflat, eaflat, eib, poole, pooln, g_all, ge1t, gn1,
      wn, bn_b[None], we, be_b[None],
      m1['wpi'], m1['wpj'], m1['wpe'], m1['pb1'], m1['pw2'], m1['pb2'],
      m1['wgx'], m1['wga'], m1['gb1'], m1['gw2'], m1['gb2'],
      m1['wno'], m1['weo'], m1['wgo'], m1['ob1'], m1['ow2'], m1['ob2'],
      m2['wpg'], m2['wgg'])

    ge2t = jnp.concatenate([ge2] * 5, axis=0)

    gam2 = pl.pallas_call(
        _mp2_body,
        grid=(NBLK,),
        in_specs=[
            pl.BlockSpec((NB, D), row2),
            pl.BlockSpec((EB, D), row2),
            _full((8, EB)),
            _full((5 * G, D)), _full((G, D)),
            _full((D, D)), _full((D, D)), _full((D, D)), _full((1, D)),
            _full((D, D)), _full((1, D)),
            _full((D, D)), _full((D, D)), _full((1, D)), _full((D, D)),
            _full((1, D)),
        ],
        out_specs=[pl.BlockSpec((NB, D), row2)],
        out_shape=[jax.ShapeDtypeStruct((G * NN, D), f32)],
    )(gam1, phi1, eib, ge2t, gn2,
      m2['wpi'], m2['wpj'], m2['wpe'], m2['pb1'], m2['pw2'], m2['pb2'],
      m2['wgx'], m2['wga'], m2['gb1'], m2['gw2'], m2['gb2'])[0]

    lp = params['lstm']
    seq = gam2.reshape(T, NN * B, D)
    bias = (lp['bih'] + lp['bhh'])[None]
    h = pl.pallas_call(
        _lstm_body,
        grid=(1,),
        in_specs=[_full((T, NN * B, D)), _full((4 * D, D)), _full((4 * D, D)),
                  _full((1, 4 * D))],
        out_specs=[_full((NN * B, D))],
        out_shape=[jax.ShapeDtypeStruct((NN * B, D), f32)],
    )(seq, lp['Wih'], lp['Whh'], bias)[0]

    return h.reshape(B, NN, D)
